# baseline (speedup 1.0000x reference)
import functools

import jax
import jax.numpy as jnp
from jax import lax
from jax.experimental import pallas as pl
from jax.experimental.pallas import tpu as pltpu
from jax.experimental.pallas import tpu_sc as plsc

N = 10000
E = 320000
F = 128
NC = 6

NUM_CORES = 2
NUM_SUBCORES = 16
NUM_WORKERS = NUM_CORES * NUM_SUBCORES

CHUNK = 128
CPW = 80
EPW = CPW * CHUNK
E_PAD = NUM_WORKERS * EPW
NPAD = 10112
RPS = NPAD // NUM_SUBCORES

_mesh = plsc.VectorSubcoreMesh(core_axis_name="c", subcore_axis_name="s")


_NBUF = 2
_HCPW = CPW // 2

_agg_scratch = [
    pltpu.VMEM_SHARED((NPAD, F), jnp.float32),
    pltpu.VMEM((_HCPW, CHUNK), jnp.int32),
    pltpu.VMEM((_HCPW, CHUNK), jnp.int32),
    pltpu.VMEM((_NBUF, CHUNK, F), jnp.float32),
    pltpu.SemaphoreType.DMA,
    [pltpu.SemaphoreType.DMA] * _NBUF,
    [pltpu.SemaphoreType.DMA] * _NBUF,
]


def _fill_block(buf, value):
    val = jnp.full((16,), value, jnp.float32)

    def row(i, carry):
        for j in range(F // 16):
            buf[i, pl.ds(j * 16, 16)] = val
        return carry

    lax.fori_loop(0, CHUNK, row, None)


def _init_table(src_blk, table, s):
    left = RPS
    off = s * RPS
    while left > 0:
        n = min(left, CHUNK)
        pltpu.sync_copy(src_blk.at[pl.ds(0, n)], table.at[pl.ds(off, n)])
        off += n
        left -= n


def _sc_agg_body(h_hbm, src_hbm, dst_hbm, out_hbm,
                 acc, sidx, didx, rows, isem, gsems, ssems):
    c = lax.axis_index("c")
    s = lax.axis_index("s")
    wid = s * NUM_CORES + c
    ic = pltpu.async_copy(src_hbm.at[pl.ds(wid * CPW, _HCPW)], sidx, isem)
    _fill_block(rows.at[0], 0.0)
    _init_table(rows.at[0], acc, s)
    ic.wait()
    pltpu.sync_copy(dst_hbm.at[pl.ds(wid * CPW, _HCPW)], didx)
    plsc.subcore_barrier()

    def quad(q, carry):
        i0 = _NBUF * q
        gs = [pltpu.async_copy(h_hbm.at[sidx.at[i0 + j]], rows.at[j], gsems[j])
              for j in range(_NBUF)]
        ss = []
        for j in range(_NBUF):
            gs[j].wait()
            ss.append(pltpu.async_copy(rows.at[j], acc.at[didx.at[i0 + j]],
                                       ssems[j], add=True))
        for j in range(_NBUF):
            ss[j].wait()
        return carry

    for half in range(2):
        if half:
            pltpu.sync_copy(src_hbm.at[pl.ds(wid * CPW + _HCPW, _HCPW)], sidx)
            pltpu.sync_copy(dst_hbm.at[pl.ds(wid * CPW + _HCPW, _HCPW)], didx)
        lax.fori_loop(0, _HCPW // _NBUF, quad, None)

    plsc.subcore_barrier()
    pltpu.sync_copy(acc.at[pl.ds(s * RPS, RPS)], out_hbm.at[c, pl.ds(s * RPS, RPS)])


_sc_agg = pl.kernel(
    _sc_agg_body,
    out_type=jax.ShapeDtypeStruct((NUM_CORES, NPAD, F), jnp.float32),
    mesh=_mesh,
    scratch_types=_agg_scratch,
)


_deg_scratch = [
    pltpu.VMEM_SHARED((NPAD, F), jnp.float32),
    pltpu.VMEM((CPW, CHUNK), jnp.int32),
    pltpu.VMEM((CHUNK, F), jnp.float32),
    pltpu.SemaphoreType.DMA,
    [pltpu.SemaphoreType.DMA] * _NBUF,
]


def _sc_deg_body(dst_hbm, out_hbm, dacc, didx, ones_v, isem, ssems):
    c = lax.axis_index("c")
    s = lax.axis_index("s")
    wid = s * NUM_CORES + c
    ic = pltpu.async_copy(dst_hbm.at[pl.ds(wid * CPW, CPW)], didx, isem)
    _fill_block(ones_v, 0.0)
    _init_table(ones_v, dacc, s)
    _fill_block(ones_v, 1.0)
    ic.wait()
    plsc.subcore_barrier()

    def quad(q, carry):
        i0 = _NBUF * q
        ss = [pltpu.async_copy(ones_v, dacc.at[didx.at[i0 + j]], ssems[j],
                               add=True)
              for j in range(_NBUF)]
        for j in range(_NBUF):
            ss[j].wait()
        return carry

    lax.fori_loop(0, CPW // _NBUF, quad, None)

    plsc.subcore_barrier()
    pltpu.sync_copy(dacc.at[pl.ds(s * RPS, RPS)], out_hbm.at[c, pl.ds(s * RPS, RPS)])


_sc_deg = pl.kernel(
    _sc_deg_body,
    out_type=jax.ShapeDtypeStruct((NUM_CORES, NPAD, F), jnp.float32),
    mesh=_mesh,
    scratch_types=_deg_scratch,
)


_RB = 2000
_GRID = N // _RB


def _lin1_body(x_ref, w_ref, b_ref, o_ref):
    o_ref[...] = jnp.maximum(
        jnp.dot(x_ref[...], w_ref[...], preferred_element_type=jnp.float32)
        + b_ref[...], 0.0)


_tc_lin1 = pl.pallas_call(
    _lin1_body,
    grid=(_GRID,),
    in_specs=[
        pl.BlockSpec((_RB, F), lambda b: (b, 0)),
        pl.BlockSpec((F, F), lambda b: (0, 0)),
        pl.BlockSpec((1, F), lambda b: (0, 0)),
    ],
    out_specs=pl.BlockSpec((_RB, F), lambda b: (b, 0)),
    out_shape=jax.ShapeDtypeStruct((N, F), jnp.float32),
)


def _conv_body(h_ref, p_ref, d_ref, w_ref, b_ref, o_ref):
    h = h_ref[...]
    agg = p_ref[0] + p_ref[1]
    deg = d_ref[0, :, 0:1] + d_ref[1, :, 0:1]
    agg = agg / jnp.maximum(deg, 1.0)
    acc = jnp.dot(h, w_ref[0:F, :], preferred_element_type=jnp.float32)
    acc += jnp.dot(agg, w_ref[F:2 * F, :], preferred_element_type=jnp.float32)
    o_ref[...] = jnp.maximum(acc + b_ref[...], 0.0)


_tc_conv = pl.pallas_call(
    _conv_body,
    grid=(_GRID,),
    in_specs=[
        pl.BlockSpec((_RB, F), lambda b: (b, 0)),
        pl.BlockSpec((NUM_CORES, _RB, F), lambda b: (0, b, 0)),
        pl.BlockSpec((NUM_CORES, _RB, F), lambda b: (0, b, 0)),
        pl.BlockSpec((2 * F, F), lambda b: (0, 0)),
        pl.BlockSpec((1, F), lambda b: (0, 0)),
    ],
    out_specs=pl.BlockSpec((_RB, F), lambda b: (b, 0)),
    out_shape=jax.ShapeDtypeStruct((N, F), jnp.float32),
)


def _final_body(h_ref, g_ref, be_ref, w2_ref, b2_ref, w3_ref, b3_ref, o_ref):
    h = h_ref[...]
    mean = jnp.mean(h, axis=0, keepdims=True)
    var = jnp.mean((h - mean) ** 2, axis=0, keepdims=True)
    hb = (h - mean) * lax.rsqrt(var + 1e-5) * g_ref[...] + be_ref[...]
    h2 = jnp.maximum(
        jnp.dot(hb, w2_ref[...], preferred_element_type=jnp.float32)
        + b2_ref[...], 0.0)
    o_ref[...] = (jnp.dot(h2, w3_ref[...], preferred_element_type=jnp.float32)
                  + b3_ref[...])


_tc_final = pl.pallas_call(
    _final_body,
    out_shape=jax.ShapeDtypeStruct((N, NC), jnp.float32),
)


def kernel(x, edge_index, W_lin1, b_lin1, W_conv1, b_conv1, W_conv2, b_conv2,
           W_conv3, b_conv3, gamma, beta, W_lin2, b_lin2, W_lin3, b_lin3):
    src = edge_index[0]
    dst = edge_index[1]
    npad = E_PAD - E
    src_p = jnp.concatenate([src, jnp.zeros((npad,), jnp.int32)])
    src_p = src_p.reshape(NUM_WORKERS * CPW, CHUNK)
    dst_p = jnp.concatenate([dst, jnp.full((npad,), N, jnp.int32)])
    dst_p = dst_p.reshape(NUM_WORKERS * CPW, CHUNK)
    degp = _sc_deg(dst_p)

    h = _tc_lin1(x, W_lin1, b_lin1.reshape(1, F))
    for W, b in ((W_conv1, b_conv1), (W_conv2, b_conv2), (W_conv3, b_conv3)):
        aggp = _sc_agg(h * 1.0000001, src_p, dst_p)
        h = _tc_conv(h, aggp, degp, W, b.reshape(1, F))

    return _tc_final(h, gamma.reshape(1, F), beta.reshape(1, F),
                     W_lin2, b_lin2.reshape(1, NC),
                     W_lin3, b_lin3.reshape(1, NC))


# speedup vs baseline: 1.0825x; 1.0825x over previous
import functools

import jax
import jax.numpy as jnp
from jax import lax
from jax.experimental import pallas as pl
from jax.experimental.pallas import tpu as pltpu
from jax.experimental.pallas import tpu_sc as plsc

N = 10000
E = 320000
F = 128
NC = 6

NUM_CORES = 2
NUM_SUBCORES = 16
NUM_WORKERS = NUM_CORES * NUM_SUBCORES

CHUNK = 128
CPW = 80
EPW = CPW * CHUNK
E_PAD = NUM_WORKERS * EPW
NPAD = 10112
RPS = NPAD // NUM_SUBCORES

_mesh = plsc.VectorSubcoreMesh(core_axis_name="c", subcore_axis_name="s")


_NBUF = 2
_HCPW = CPW // 2

_agg_scratch = [
    pltpu.VMEM_SHARED((NPAD, F), jnp.float32),
    pltpu.VMEM((_HCPW, CHUNK), jnp.int32),
    pltpu.VMEM((_HCPW, CHUNK), jnp.int32),
    pltpu.VMEM((_NBUF, CHUNK, F), jnp.float32),
    pltpu.SemaphoreType.DMA,
    [pltpu.SemaphoreType.DMA] * _NBUF,
    [pltpu.SemaphoreType.DMA] * _NBUF,
]


def _fill_block(buf, value):
    val = jnp.full((16,), value, jnp.float32)

    def row(i, carry):
        for j in range(F // 16):
            buf[i, pl.ds(j * 16, 16)] = val
        return carry

    lax.fori_loop(0, CHUNK, row, None)


def _init_table(src_blk, table, s):
    left = RPS
    off = s * RPS
    while left > 0:
        n = min(left, CHUNK)
        pltpu.sync_copy(src_blk.at[pl.ds(0, n)], table.at[pl.ds(off, n)])
        off += n
        left -= n


def _sc_agg_body(h_hbm, src_hbm, dst_hbm, out_hbm,
                 acc, sidx, didx, rows, isem, gsems, ssems):
    c = lax.axis_index("c")
    s = lax.axis_index("s")
    wid = s * NUM_CORES + c
    ic = pltpu.async_copy(src_hbm.at[pl.ds(wid * CPW, _HCPW)], sidx, isem)
    _fill_block(rows.at[0], 0.0)
    _init_table(rows.at[0], acc, s)
    ic.wait()
    pltpu.sync_copy(dst_hbm.at[pl.ds(wid * CPW, _HCPW)], didx)
    plsc.subcore_barrier()

    def quad(q, carry):
        i0 = _NBUF * q
        gs = [pltpu.async_copy(h_hbm.at[sidx.at[i0 + j]], rows.at[j], gsems[j])
              for j in range(_NBUF)]
        ss = []
        for j in range(_NBUF):
            gs[j].wait()
            ss.append(pltpu.async_copy(rows.at[j], acc.at[didx.at[i0 + j]],
                                       ssems[j], add=True))
        for j in range(_NBUF):
            ss[j].wait()
        return carry

    for half in range(2):
        if half:
            pltpu.sync_copy(src_hbm.at[pl.ds(wid * CPW + _HCPW, _HCPW)], sidx)
            pltpu.sync_copy(dst_hbm.at[pl.ds(wid * CPW + _HCPW, _HCPW)], didx)
        lax.fori_loop(0, _HCPW // _NBUF, quad, None)

    plsc.subcore_barrier()
    pltpu.sync_copy(acc.at[pl.ds(s * RPS, RPS)], out_hbm.at[c, pl.ds(s * RPS, RPS)])


_sc_agg = pl.kernel(
    _sc_agg_body,
    out_type=jax.ShapeDtypeStruct((NUM_CORES, NPAD, F), jnp.float32),
    mesh=_mesh,
    scratch_types=_agg_scratch,
)


_deg_scratch = [
    pltpu.VMEM_SHARED((NPAD, F), jnp.float32),
    pltpu.VMEM((CPW, CHUNK), jnp.int32),
    pltpu.VMEM((CHUNK, F), jnp.float32),
    pltpu.SemaphoreType.DMA,
    [pltpu.SemaphoreType.DMA] * _NBUF,
]


def _sc_deg_body(dst_hbm, out_hbm, dacc, didx, ones_v, isem, ssems):
    c = lax.axis_index("c")
    s = lax.axis_index("s")
    wid = s * NUM_CORES + c
    ic = pltpu.async_copy(dst_hbm.at[pl.ds(wid * CPW, CPW)], didx, isem)
    _fill_block(ones_v, 0.0)
    _init_table(ones_v, dacc, s)
    _fill_block(ones_v, 1.0)
    ic.wait()
    plsc.subcore_barrier()

    def quad(q, carry):
        i0 = _NBUF * q
        ss = [pltpu.async_copy(ones_v, dacc.at[didx.at[i0 + j]], ssems[j],
                               add=True)
              for j in range(_NBUF)]
        for j in range(_NBUF):
            ss[j].wait()
        return carry

    lax.fori_loop(0, CPW // _NBUF, quad, None)

    plsc.subcore_barrier()
    pltpu.sync_copy(dacc.at[pl.ds(s * RPS, RPS)], out_hbm.at[c, pl.ds(s * RPS, RPS)])


_sc_deg = pl.kernel(
    _sc_deg_body,
    out_type=jax.ShapeDtypeStruct((NUM_CORES, NPAD, F), jnp.float32),
    mesh=_mesh,
    scratch_types=_deg_scratch,
)


_RB = 2000
_GRID = N // _RB


def _lin1_body(x_ref, w_ref, b_ref, o_ref):
    o_ref[...] = jnp.maximum(
        jnp.dot(x_ref[...], w_ref[...], preferred_element_type=jnp.float32)
        + b_ref[...], 0.0)


_tc_lin1 = pl.pallas_call(
    _lin1_body,
    grid=(_GRID,),
    in_specs=[
        pl.BlockSpec((_RB, F), lambda b: (b, 0)),
        pl.BlockSpec((F, F), lambda b: (0, 0)),
        pl.BlockSpec((1, F), lambda b: (0, 0)),
    ],
    out_specs=pl.BlockSpec((_RB, F), lambda b: (b, 0)),
    out_shape=jax.ShapeDtypeStruct((N, F), jnp.float32),
)


def _conv_body(h_ref, p_ref, d_ref, w_ref, b_ref, o_ref):
    h = h_ref[...]
    agg = p_ref[0] + p_ref[1]
    deg = d_ref[0, :, 0:1] + d_ref[1, :, 0:1]
    agg = agg / jnp.maximum(deg, 1.0)
    acc = jnp.dot(h, w_ref[0:F, :], preferred_element_type=jnp.float32)
    acc += jnp.dot(agg, w_ref[F:2 * F, :], preferred_element_type=jnp.float32)
    o_ref[...] = jnp.maximum(acc + b_ref[...], 0.0)


_tc_conv = pl.pallas_call(
    _conv_body,
    grid=(_GRID,),
    in_specs=[
        pl.BlockSpec((_RB, F), lambda b: (b, 0)),
        pl.BlockSpec((NUM_CORES, _RB, F), lambda b: (0, b, 0)),
        pl.BlockSpec((NUM_CORES, _RB, F), lambda b: (0, b, 0)),
        pl.BlockSpec((2 * F, F), lambda b: (0, 0)),
        pl.BlockSpec((1, F), lambda b: (0, 0)),
    ],
    out_specs=pl.BlockSpec((_RB, F), lambda b: (b, 0)),
    out_shape=jax.ShapeDtypeStruct((N, F), jnp.float32),
)


def _final_body(h_ref, g_ref, be_ref, w2_ref, b2_ref, w3_ref, b3_ref, o_ref):
    h = h_ref[...]
    mean = jnp.mean(h, axis=0, keepdims=True)
    var = jnp.mean((h - mean) ** 2, axis=0, keepdims=True)
    hb = (h - mean) * lax.rsqrt(var + 1e-5) * g_ref[...] + be_ref[...]
    h2 = jnp.maximum(
        jnp.dot(hb, w2_ref[...], preferred_element_type=jnp.float32)
        + b2_ref[...], 0.0)
    o_ref[...] = (jnp.dot(h2, w3_ref[...], preferred_element_type=jnp.float32)
                  + b3_ref[...])


_tc_final = pl.pallas_call(
    _final_body,
    out_shape=jax.ShapeDtypeStruct((N, NC), jnp.float32),
)


def kernel(x, edge_index, W_lin1, b_lin1, W_conv1, b_conv1, W_conv2, b_conv2,
           W_conv3, b_conv3, gamma, beta, W_lin2, b_lin2, W_lin3, b_lin3):
    src = edge_index[0]
    dst = edge_index[1]
    npad = E_PAD - E
    src_p = jnp.concatenate([src, jnp.zeros((npad,), jnp.int32)])
    src_p = src_p.reshape(NUM_WORKERS * CPW, CHUNK)
    dst_p = jnp.concatenate([dst, jnp.full((npad,), N, jnp.int32)])
    dst_p = dst_p.reshape(NUM_WORKERS * CPW, CHUNK)
    degp = _sc_deg(dst_p)

    h = _tc_lin1(x, W_lin1, b_lin1.reshape(1, F))
    for W, b in ((W_conv1, b_conv1), (W_conv2, b_conv2), (W_conv3, b_conv3)):
        aggp = _sc_agg(h, src_p, dst_p)
        h = _tc_conv(h, aggp, degp, W, b.reshape(1, F))

    return _tc_final(h, gamma.reshape(1, F), beta.reshape(1, F),
                     W_lin2, b_lin2.reshape(1, NC),
                     W_lin3, b_lin3.reshape(1, NC))


# speedup vs baseline: 1.2260x; 1.1326x over previous
import functools

import jax
import jax.numpy as jnp
from jax import lax
from jax.experimental import pallas as pl
from jax.experimental.pallas import tpu as pltpu
from jax.experimental.pallas import tpu_sc as plsc

N = 10000
E = 320000
F = 128
NC = 6

NUM_CORES = 2
NUM_SUBCORES = 16
NUM_WORKERS = NUM_CORES * NUM_SUBCORES

CHUNK = 128
CPW = 80
EPW = CPW * CHUNK
E_PAD = NUM_WORKERS * EPW
NPAD = 10112
RPS = NPAD // NUM_SUBCORES

_mesh = plsc.VectorSubcoreMesh(core_axis_name="c", subcore_axis_name="s")


_NBUF = 2
_HCPW = CPW // 2

CPW0 = 120
CPW1 = 40
_STAGE = 40

_agg_scratch = [
    pltpu.VMEM_SHARED((NPAD, F), jnp.float32),
    pltpu.VMEM((_STAGE, CHUNK), jnp.int32),
    pltpu.VMEM((_STAGE, CHUNK), jnp.int32),
    pltpu.VMEM((_NBUF, CHUNK, F), jnp.float32),
    pltpu.SemaphoreType.DMA,
    [pltpu.SemaphoreType.DMA] * _NBUF,
    [pltpu.SemaphoreType.DMA] * _NBUF,
]


def _fill_block(buf, value):
    val = jnp.full((16,), value, jnp.float32)

    def row(i, carry):
        for j in range(F // 16):
            buf[i, pl.ds(j * 16, 16)] = val
        return carry

    lax.fori_loop(0, CHUNK, row, None)


def _init_table(src_blk, table, s):
    left = RPS
    off = s * RPS
    while left > 0:
        n = min(left, CHUNK)
        pltpu.sync_copy(src_blk.at[pl.ds(0, n)], table.at[pl.ds(off, n)])
        off += n
        left -= n


def _sc_agg_body(h_hbm, src_hbm, dst_hbm, out_hbm,
                 acc, sidx, didx, rows, isem, gsems, ssems):
    c = lax.axis_index("c")
    s = lax.axis_index("s")
    _fill_block(rows.at[0], 0.0)
    _init_table(rows.at[0], acc, s)
    plsc.subcore_barrier()

    def quad(q, carry):
        i0 = _NBUF * q
        gs = [pltpu.async_copy(h_hbm.at[sidx.at[i0 + j]], rows.at[j], gsems[j])
              for j in range(_NBUF)]
        ss = []
        for j in range(_NBUF):
            gs[j].wait()
            ss.append(pltpu.async_copy(rows.at[j], acc.at[didx.at[i0 + j]],
                                       ssems[j], add=True))
        for j in range(_NBUF):
            ss[j].wait()
        return carry

    def run(cpw, base_chunk):
        for stage in range(cpw // _STAGE):
            off = base_chunk + stage * _STAGE
            pltpu.sync_copy(src_hbm.at[pl.ds(off, _STAGE)], sidx)
            pltpu.sync_copy(dst_hbm.at[pl.ds(off, _STAGE)], didx)
            lax.fori_loop(0, _STAGE // _NBUF, quad, None)

    @pl.when(c == 0)
    def _():
        run(CPW0, s * CPW0)

    @pl.when(c == 1)
    def _():
        run(CPW1, NUM_SUBCORES * CPW0 + s * CPW1)

    plsc.subcore_barrier()
    pltpu.sync_copy(acc.at[pl.ds(s * RPS, RPS)], out_hbm.at[c, pl.ds(s * RPS, RPS)])


_sc_agg = pl.kernel(
    _sc_agg_body,
    out_type=jax.ShapeDtypeStruct((NUM_CORES, NPAD, F), jnp.float32),
    mesh=_mesh,
    scratch_types=_agg_scratch,
)


_deg_scratch = [
    pltpu.VMEM_SHARED((NPAD, F), jnp.float32),
    pltpu.VMEM((CPW, CHUNK), jnp.int32),
    pltpu.VMEM((CHUNK, F), jnp.float32),
    pltpu.SemaphoreType.DMA,
    [pltpu.SemaphoreType.DMA] * _NBUF,
]


def _sc_deg_body(dst_hbm, out_hbm, dacc, didx, ones_v, isem, ssems):
    c = lax.axis_index("c")
    s = lax.axis_index("s")
    wid = s * NUM_CORES + c
    ic = pltpu.async_copy(dst_hbm.at[pl.ds(wid * CPW, CPW)], didx, isem)
    _fill_block(ones_v, 0.0)
    _init_table(ones_v, dacc, s)
    _fill_block(ones_v, 1.0)
    ic.wait()
    plsc.subcore_barrier()

    def quad(q, carry):
        i0 = _NBUF * q
        ss = [pltpu.async_copy(ones_v, dacc.at[didx.at[i0 + j]], ssems[j],
                               add=True)
              for j in range(_NBUF)]
        for j in range(_NBUF):
            ss[j].wait()
        return carry

    lax.fori_loop(0, CPW // _NBUF, quad, None)

    plsc.subcore_barrier()
    pltpu.sync_copy(dacc.at[pl.ds(s * RPS, RPS)], out_hbm.at[c, pl.ds(s * RPS, RPS)])


_sc_deg = pl.kernel(
    _sc_deg_body,
    out_type=jax.ShapeDtypeStruct((NUM_CORES, NPAD, F), jnp.float32),
    mesh=_mesh,
    scratch_types=_deg_scratch,
)


_RB = 2000
_GRID = N // _RB


def _lin1_body(x_ref, w_ref, b_ref, o_ref):
    o_ref[...] = jnp.maximum(
        jnp.dot(x_ref[...], w_ref[...], preferred_element_type=jnp.float32)
        + b_ref[...], 0.0)


_tc_lin1 = pl.pallas_call(
    _lin1_body,
    grid=(_GRID,),
    in_specs=[
        pl.BlockSpec((_RB, F), lambda b: (b, 0)),
        pl.BlockSpec((F, F), lambda b: (0, 0)),
        pl.BlockSpec((1, F), lambda b: (0, 0)),
    ],
    out_specs=pl.BlockSpec((_RB, F), lambda b: (b, 0)),
    out_shape=jax.ShapeDtypeStruct((N, F), jnp.float32),
)


def _conv_body(h_ref, p_ref, d_ref, w_ref, b_ref, o_ref):
    h = h_ref[...]
    agg = p_ref[0] + p_ref[1]
    deg = d_ref[0, :, 0:1] + d_ref[1, :, 0:1]
    agg = agg / jnp.maximum(deg, 1.0)
    acc = jnp.dot(h, w_ref[0:F, :], preferred_element_type=jnp.float32)
    acc += jnp.dot(agg, w_ref[F:2 * F, :], preferred_element_type=jnp.float32)
    o_ref[...] = jnp.maximum(acc + b_ref[...], 0.0)


_tc_conv = pl.pallas_call(
    _conv_body,
    grid=(_GRID,),
    in_specs=[
        pl.BlockSpec((_RB, F), lambda b: (b, 0)),
        pl.BlockSpec((NUM_CORES, _RB, F), lambda b: (0, b, 0)),
        pl.BlockSpec((NUM_CORES, _RB, F), lambda b: (0, b, 0)),
        pl.BlockSpec((2 * F, F), lambda b: (0, 0)),
        pl.BlockSpec((1, F), lambda b: (0, 0)),
    ],
    out_specs=pl.BlockSpec((_RB, F), lambda b: (b, 0)),
    out_shape=jax.ShapeDtypeStruct((N, F), jnp.float32),
)


def _final_body(h_ref, g_ref, be_ref, w2_ref, b2_ref, w3_ref, b3_ref, o_ref):
    h = h_ref[...]
    mean = jnp.mean(h, axis=0, keepdims=True)
    var = jnp.mean((h - mean) ** 2, axis=0, keepdims=True)
    hb = (h - mean) * lax.rsqrt(var + 1e-5) * g_ref[...] + be_ref[...]
    h2 = jnp.maximum(
        jnp.dot(hb, w2_ref[...], preferred_element_type=jnp.float32)
        + b2_ref[...], 0.0)
    o_ref[...] = (jnp.dot(h2, w3_ref[...], preferred_element_type=jnp.float32)
                  + b3_ref[...])


_tc_final = pl.pallas_call(
    _final_body,
    out_shape=jax.ShapeDtypeStruct((N, NC), jnp.float32),
)


def kernel(x, edge_index, W_lin1, b_lin1, W_conv1, b_conv1, W_conv2, b_conv2,
           W_conv3, b_conv3, gamma, beta, W_lin2, b_lin2, W_lin3, b_lin3):
    src = edge_index[0]
    dst = edge_index[1]
    npad = E_PAD - E
    src_p = jnp.concatenate([src, jnp.zeros((npad,), jnp.int32)])
    src_p = src_p.reshape(NUM_WORKERS * CPW, CHUNK)
    dst_p = jnp.concatenate([dst, jnp.full((npad,), N, jnp.int32)])
    dst_p = dst_p.reshape(NUM_WORKERS * CPW, CHUNK)
    degp = _sc_deg(dst_p)

    h = _tc_lin1(x, W_lin1, b_lin1.reshape(1, F))
    for W, b in ((W_conv1, b_conv1), (W_conv2, b_conv2), (W_conv3, b_conv3)):
        aggp = _sc_agg(h, src_p, dst_p)
        h = _tc_conv(h, aggp, degp, W, b.reshape(1, F))

    return _tc_final(h, gamma.reshape(1, F), beta.reshape(1, F),
                     W_lin2, b_lin2.reshape(1, NC),
                     W_lin3, b_lin3.reshape(1, NC))


# speedup vs baseline: 1.2749x; 1.0399x over previous
import functools

import jax
import jax.numpy as jnp
from jax import lax
from jax.experimental import pallas as pl
from jax.experimental.pallas import tpu as pltpu
from jax.experimental.pallas import tpu_sc as plsc

N = 10000
E = 320000
F = 128
NC = 6

NUM_CORES = 2
NUM_SUBCORES = 16
NUM_WORKERS = NUM_CORES * NUM_SUBCORES

CHUNK = 128
CPW = 80
EPW = CPW * CHUNK
E_PAD = NUM_WORKERS * EPW
NPAD = 10112
RPS = NPAD // NUM_SUBCORES

_mesh = plsc.VectorSubcoreMesh(core_axis_name="c", subcore_axis_name="s")


_NBUF = 2
_HCPW = CPW // 2

CPW0 = 136
CPW1 = 24
_STAGES0 = (40, 40, 40, 16)
_STAGES1 = (24,)
_STAGE = 40

_agg_scratch = [
    pltpu.VMEM_SHARED((NPAD, F), jnp.float32),
    pltpu.VMEM((_STAGE, CHUNK), jnp.int32),
    pltpu.VMEM((_STAGE, CHUNK), jnp.int32),
    pltpu.VMEM((_NBUF, CHUNK, F), jnp.float32),
    pltpu.SemaphoreType.DMA,
    [pltpu.SemaphoreType.DMA] * _NBUF,
    [pltpu.SemaphoreType.DMA] * _NBUF,
]


def _fill_block(buf, value):
    val = jnp.full((16,), value, jnp.float32)

    def row(i, carry):
        for j in range(F // 16):
            buf[i, pl.ds(j * 16, 16)] = val
        return carry

    lax.fori_loop(0, CHUNK, row, None)


def _init_table(src_blk, table, s):
    left = RPS
    off = s * RPS
    while left > 0:
        n = min(left, CHUNK)
        pltpu.sync_copy(src_blk.at[pl.ds(0, n)], table.at[pl.ds(off, n)])
        off += n
        left -= n


def _sc_agg_body(h_hbm, src_hbm, dst_hbm, out_hbm,
                 acc, sidx, didx, rows, isem, gsems, ssems):
    c = lax.axis_index("c")
    s = lax.axis_index("s")
    _fill_block(rows.at[0], 0.0)
    _init_table(rows.at[0], acc, s)
    plsc.subcore_barrier()

    def quad(q, carry):
        i0 = _NBUF * q
        gs = [pltpu.async_copy(h_hbm.at[sidx.at[i0 + j]], rows.at[j], gsems[j])
              for j in range(_NBUF)]
        ss = []
        for j in range(_NBUF):
            gs[j].wait()
            ss.append(pltpu.async_copy(rows.at[j], acc.at[didx.at[i0 + j]],
                                       ssems[j], add=True))
        for j in range(_NBUF):
            ss[j].wait()
        return carry

    def run(stages, base_chunk):
        off = base_chunk
        for n in stages:
            pltpu.sync_copy(src_hbm.at[pl.ds(off, n)], sidx.at[pl.ds(0, n)])
            pltpu.sync_copy(dst_hbm.at[pl.ds(off, n)], didx.at[pl.ds(0, n)])
            lax.fori_loop(0, n // _NBUF, quad, None)
            off += n

    @pl.when(c == 0)
    def _():
        run(_STAGES0, s * CPW0)

    @pl.when(c == 1)
    def _():
        run(_STAGES1, NUM_SUBCORES * CPW0 + s * CPW1)

    plsc.subcore_barrier()
    pltpu.sync_copy(acc.at[pl.ds(s * RPS, RPS)], out_hbm.at[c, pl.ds(s * RPS, RPS)])


_sc_agg = pl.kernel(
    _sc_agg_body,
    out_type=jax.ShapeDtypeStruct((NUM_CORES, NPAD, F), jnp.float32),
    mesh=_mesh,
    scratch_types=_agg_scratch,
)


_deg_scratch = [
    pltpu.VMEM_SHARED((NPAD, F), jnp.float32),
    pltpu.VMEM((CPW, CHUNK), jnp.int32),
    pltpu.VMEM((CHUNK, F), jnp.float32),
    pltpu.SemaphoreType.DMA,
    [pltpu.SemaphoreType.DMA] * _NBUF,
]


def _sc_deg_body(dst_hbm, out_hbm, dacc, didx, ones_v, isem, ssems):
    c = lax.axis_index("c")
    s = lax.axis_index("s")
    wid = s * NUM_CORES + c
    ic = pltpu.async_copy(dst_hbm.at[pl.ds(wid * CPW, CPW)], didx, isem)
    _fill_block(ones_v, 0.0)
    _init_table(ones_v, dacc, s)
    _fill_block(ones_v, 1.0)
    ic.wait()
    plsc.subcore_barrier()

    def quad(q, carry):
        i0 = _NBUF * q
        ss = [pltpu.async_copy(ones_v, dacc.at[didx.at[i0 + j]], ssems[j],
                               add=True)
              for j in range(_NBUF)]
        for j in range(_NBUF):
            ss[j].wait()
        return carry

    lax.fori_loop(0, CPW // _NBUF, quad, None)

    plsc.subcore_barrier()
    pltpu.sync_copy(dacc.at[pl.ds(s * RPS, RPS)], out_hbm.at[c, pl.ds(s * RPS, RPS)])


_sc_deg = pl.kernel(
    _sc_deg_body,
    out_type=jax.ShapeDtypeStruct((NUM_CORES, NPAD, F), jnp.float32),
    mesh=_mesh,
    scratch_types=_deg_scratch,
)


_RB = 2000
_GRID = N // _RB


def _lin1_body(x_ref, w_ref, b_ref, o_ref):
    o_ref[...] = jnp.maximum(
        jnp.dot(x_ref[...], w_ref[...], preferred_element_type=jnp.float32)
        + b_ref[...], 0.0)


_tc_lin1 = pl.pallas_call(
    _lin1_body,
    grid=(_GRID,),
    in_specs=[
        pl.BlockSpec((_RB, F), lambda b: (b, 0)),
        pl.BlockSpec((F, F), lambda b: (0, 0)),
        pl.BlockSpec((1, F), lambda b: (0, 0)),
    ],
    out_specs=pl.BlockSpec((_RB, F), lambda b: (b, 0)),
    out_shape=jax.ShapeDtypeStruct((N, F), jnp.float32),
)


def _conv_body(h_ref, p_ref, d_ref, w_ref, b_ref, o_ref):
    h = h_ref[...]
    agg = p_ref[0] + p_ref[1]
    deg = d_ref[0, :, 0:1] + d_ref[1, :, 0:1]
    agg = agg / jnp.maximum(deg, 1.0)
    acc = jnp.dot(h, w_ref[0:F, :], preferred_element_type=jnp.float32)
    acc += jnp.dot(agg, w_ref[F:2 * F, :], preferred_element_type=jnp.float32)
    o_ref[...] = jnp.maximum(acc + b_ref[...], 0.0)


_tc_conv = pl.pallas_call(
    _conv_body,
    grid=(_GRID,),
    in_specs=[
        pl.BlockSpec((_RB, F), lambda b: (b, 0)),
        pl.BlockSpec((NUM_CORES, _RB, F), lambda b: (0, b, 0)),
        pl.BlockSpec((NUM_CORES, _RB, F), lambda b: (0, b, 0)),
        pl.BlockSpec((2 * F, F), lambda b: (0, 0)),
        pl.BlockSpec((1, F), lambda b: (0, 0)),
    ],
    out_specs=pl.BlockSpec((_RB, F), lambda b: (b, 0)),
    out_shape=jax.ShapeDtypeStruct((N, F), jnp.float32),
)


def _final_body(h_ref, g_ref, be_ref, w2_ref, b2_ref, w3_ref, b3_ref, o_ref):
    h = h_ref[...]
    mean = jnp.mean(h, axis=0, keepdims=True)
    var = jnp.mean((h - mean) ** 2, axis=0, keepdims=True)
    hb = (h - mean) * lax.rsqrt(var + 1e-5) * g_ref[...] + be_ref[...]
    h2 = jnp.maximum(
        jnp.dot(hb, w2_ref[...], preferred_element_type=jnp.float32)
        + b2_ref[...], 0.0)
    o_ref[...] = (jnp.dot(h2, w3_ref[...], preferred_element_type=jnp.float32)
                  + b3_ref[...])


_tc_final = pl.pallas_call(
    _final_body,
    out_shape=jax.ShapeDtypeStruct((N, NC), jnp.float32),
)


def kernel(x, edge_index, W_lin1, b_lin1, W_conv1, b_conv1, W_conv2, b_conv2,
           W_conv3, b_conv3, gamma, beta, W_lin2, b_lin2, W_lin3, b_lin3):
    src = edge_index[0]
    dst = edge_index[1]
    npad = E_PAD - E
    src_p = jnp.concatenate([src, jnp.zeros((npad,), jnp.int32)])
    src_p = src_p.reshape(NUM_WORKERS * CPW, CHUNK)
    dst_p = jnp.concatenate([dst, jnp.full((npad,), N, jnp.int32)])
    dst_p = dst_p.reshape(NUM_WORKERS * CPW, CHUNK)
    degp = _sc_deg(dst_p)

    h = _tc_lin1(x, W_lin1, b_lin1.reshape(1, F))
    for W, b in ((W_conv1, b_conv1), (W_conv2, b_conv2), (W_conv3, b_conv3)):
        aggp = _sc_agg(h, src_p, dst_p)
        h = _tc_conv(h, aggp, degp, W, b.reshape(1, F))

    return _tc_final(h, gamma.reshape(1, F), beta.reshape(1, F),
                     W_lin2, b_lin2.reshape(1, NC),
                     W_lin3, b_lin3.reshape(1, NC))
